# baseline (speedup 1.0000x reference)
import jax
import jax.numpy as jnp
from jax import lax
from jax.experimental import pallas as pl
from jax.experimental.pallas import tpu as pltpu
from jax.experimental.pallas import tpu_sc as plsc

NU = 25000
NI = 25000
D = 64
EH = 400000
NS = 16
NC = 2
CW = 112
EPS = EH // NS
GW = 14
NG = 16
NJ = NG * GW
ESP = NJ * CW
RS = 1568
RT = NS * RS
EB = 112
DUMMY = NU
NRB = 4
BATCH = 4096
TD = 64
MTD = 128

_f32 = jnp.float32
_bf16 = jnp.bfloat16
_i32 = jnp.int32


def _mesh():
    return plsc.VectorSubcoreMesh(core_axis_name="c", subcore_axis_name="s")


def _layer_body(with_deg, refs):
    if with_deg:
        (dstu4, srci4, dsti4, srcu4, tab_u, tab_i,
         out_u, out_i, inv_u, inv_i,
         dw0, dw1, sw0, sw1, r0, r1, r2, r3, invblk, tabsp, acc, degacc,
         sid0, sid1, sis0, sis1, sg0, sg1, sg2, sg3, ss0, ss1, ss2, ss3,
         sd0, sd1, sd2, sd3) = refs
        invin_u = invin_i = None
    else:
        (dstu4, srci4, dsti4, srcu4, tab_u, tab_i, invin_u, invin_i,
         out_u, out_i,
         dw0, dw1, sw0, sw1, r0, r1, r2, r3, invblk, tabsp, acc, degacc,
         sid0, sid1, sis0, sis1, sg0, sg1, sg2, sg3, ss0, ss1, ss2, ss3,
         sd0, sd1, sd2, sd3) = refs
        inv_u = inv_i = None
    dw = [dw0, dw1]
    sw = [sw0, sw1]
    rows = [r0, r1, r2, r3]
    sidx = [sid0, sid1]
    sis = [sis0, sis1]
    sg = [sg0, sg1, sg2, sg3]
    ss = [ss0, ss1, ss2, ss3]
    sd = [sd0, sd1, sd2, sd3]

    cid = lax.axis_index("c")
    sid = lax.axis_index("s")

    def half(dst4, src4, table, out_e, inv_out, inv_in):
        base = sid * RS
        z32 = jnp.zeros((32,), _bf16)
        zf = jnp.zeros((16,), _f32)

        pltpu.sync_copy(table.at[pl.ds(base, RS)], tabsp.at[pl.ds(base, RS)])

        @pl.loop(0, EB)
        def _(r):
            for c in range(D // 32):
                rows[0][r, pl.ds(c * 32, 32)] = z32

        for c in range(128 // 16):
            invblk[pl.ds(c * 16, 16)] = zf

        for k in range(RS // EB):
            pltpu.sync_copy(rows[0], acc.at[pl.ds(base + k * EB, EB)])
        if with_deg:
            for k in range(RS // EB):
                pltpu.sync_copy(invblk.at[pl.ds(0, EB)],
                                degacc.at[pl.ds(base + k * EB, EB)])
            for c in range(CW // 16):
                invblk[pl.ds(c * 16, 16)] = jnp.full((16,), 1.0, _f32)

        plsc.subcore_barrier()

        def idx_start(g, b):
            pltpu.async_copy(dst4.at[sid, g], dw[b], sidx[b])
            pltpu.async_copy(src4.at[sid, g], sw[b], sis[b])

        def idx_wait(b):
            pltpu.make_async_copy(dst4.at[sid, 0], dw[b], sidx[b]).wait()
            pltpu.make_async_copy(src4.at[sid, 0], sw[b], sis[b]).wait()

        def gather_start(wb, jj, b):
            pltpu.async_copy(tabsp.at[sw[wb].at[jj]], rows[b], sg[b])

        def gather_wait(b):
            pltpu.make_async_copy(tabsp.at[pl.ds(0, CW)], rows[b],
                                  sg[b]).wait()

        def scat_start(wb, jj, b):
            pltpu.async_copy(rows[b], acc.at[dw[wb].at[jj]], ss[b], add=True)
            if with_deg:
                pltpu.async_copy(invblk.at[pl.ds(0, CW)],
                                 degacc.at[dw[wb].at[jj]], sd[b], add=True)

        def scat_wait(b):
            pltpu.make_async_copy(rows[b], acc.at[pl.ds(0, CW)],
                                  ss[b]).wait()
            if with_deg:
                pltpu.make_async_copy(invblk.at[pl.ds(0, CW)],
                                      degacc.at[pl.ds(0, CW)], sd[b]).wait()

        def do_group(wb, first, prefetch):
            idx_wait(wb)
            for p in range(3):
                if not first:
                    scat_wait(p)
                gather_start(wb, p, p)
            for jj in range(GW):
                b = jj % NRB
                gather_wait(b)
                scat_start(wb, jj, b)
                if jj + 3 < GW:
                    bn = (jj + 3) % NRB
                    if not (first and jj < 1):
                        scat_wait(bn)
                    if jj == 0:
                        prefetch()
                    gather_start(wb, jj + 3, bn)

        idx_start(0, 0)
        do_group(0, True, lambda: idx_start(1, 1))
        do_group(1, False, lambda: idx_start(2, 0))

        @pl.loop(1, NG // 2)
        def _(t):
            def pf_a():
                idx_start(2 * t + 1, 1)

            def pf_b():
                @pl.when(2 * t + 2 < NG)
                def _():
                    idx_start(2 * t + 2, 0)

            do_group(0, False, pf_a)
            do_group(1, False, pf_b)

        for b in range(NRB):
            scat_wait(b)

        plsc.subcore_barrier()

        for blk in range(RS // EB):
            off = base + blk * EB
            if with_deg:
                pltpu.sync_copy(degacc.at[pl.ds(off, EB)],
                                invblk.at[pl.ds(0, EB)])

                @pl.loop(0, EB // 16)
                def _(k):
                    v = invblk[pl.ds(k * 16, 16)]
                    invblk[pl.ds(k * 16, 16)] = 1.0 / jnp.maximum(v, 1.0)

                pltpu.sync_copy(invblk.at[pl.ds(0, EB)],
                                inv_out.at[pl.ds(off, EB)])
            else:
                pltpu.sync_copy(inv_in.at[pl.ds(off, EB)],
                                invblk.at[pl.ds(0, EB)])

            pltpu.sync_copy(acc.at[pl.ds(off, EB)], rows[0])

            @pl.loop(0, EB // 16)
            def _(k):
                dv = invblk[pl.ds(k * 16, 16)]
                for q in range(16):
                    r = k * 16 + q
                    vs = jnp.full((16,), dv[q], _f32)
                    sb = plsc.pack(vs, vs, format=plsc.PackFormat.INTERLEAVED)
                    for c in range(D // 32):
                        sl = pl.ds(c * 32, 32)
                        rows[0][r, sl] = rows[0][r, sl] * sb

            pltpu.sync_copy(rows[0], out_e.at[pl.ds(off, EB)])

    @pl.when(cid == 0)
    def _():
        half(dstu4, srci4, tab_i, out_u, inv_u, invin_u)

    @pl.when(cid == 1)
    def _():
        half(dsti4, srcu4, tab_u, out_i, inv_i, invin_i)


def _layer_scratch():
    return [
        pltpu.VMEM((GW, CW), _i32),
        pltpu.VMEM((GW, CW), _i32),
        pltpu.VMEM((GW, CW), _i32),
        pltpu.VMEM((GW, CW), _i32),
        pltpu.VMEM((EB, D), _bf16),
        pltpu.VMEM((EB, D), _bf16),
        pltpu.VMEM((EB, D), _bf16),
        pltpu.VMEM((EB, D), _bf16),
        pltpu.VMEM((128,), _f32),
        pltpu.VMEM_SHARED((RT, D), _bf16),
        pltpu.VMEM_SHARED((RT, D), _bf16),
        pltpu.VMEM_SHARED((RT,), _f32),
    ] + [pltpu.SemaphoreType.DMA] * 16


def _make_layer_a():
    out_type = [
        jax.ShapeDtypeStruct((RT, D), _bf16),
        jax.ShapeDtypeStruct((RT, D), _bf16),
        jax.ShapeDtypeStruct((RT,), _f32),
        jax.ShapeDtypeStruct((RT,), _f32),
    ]
    return pl.kernel(
        lambda *refs: _layer_body(True, refs),
        out_type=out_type, mesh=_mesh(), scratch_types=_layer_scratch(),
        compiler_params=pltpu.CompilerParams(use_tc_tiling_on_sc=False,
                                             needs_layout_passes=False),
        name="gnn_layer1")


def _make_layer_b():
    out_type = [
        jax.ShapeDtypeStruct((RT, D), _bf16),
        jax.ShapeDtypeStruct((RT, D), _bf16),
    ]
    return pl.kernel(
        lambda *refs: _layer_body(False, refs),
        out_type=out_type, mesh=_mesh(), scratch_types=_layer_scratch(),
        compiler_params=pltpu.CompilerParams(use_tc_tiling_on_sc=False,
                                             needs_layout_passes=False),
        name="gnn_layer2")


def _gather_body(refs):
    (u_idx, i_idx, e0u, e1u, e2u, e0i, e1i, e2i, utw_t, itw_t,
     ue0, ue1, ue2, ie0, ie1, ie2, utwr, itwr,
     idxu, idxi, b0, b1, b2, bt, s0, s1, s2, s3) = refs
    cid = lax.axis_index("c")
    sid = lax.axis_index("s")
    wid = cid * NS + sid
    base = wid * 128
    pltpu.sync_copy(u_idx.at[pl.ds(base, 128)], idxu)
    pltpu.sync_copy(i_idx.at[pl.ds(base, 128)], idxi)

    def side(idxv, t0, t1, t2, tt, o0, o1, o2, o_t):
        g0 = pltpu.async_copy(t0.at[idxv], b0, s0)
        g1 = pltpu.async_copy(t1.at[idxv], b1, s1)
        g2 = pltpu.async_copy(t2.at[idxv], b2, s2)
        g3 = pltpu.async_copy(tt.at[idxv], bt, s3)
        g0.wait()
        pltpu.sync_copy(b0, o0.at[pl.ds(base, 128)])
        g1.wait()
        pltpu.sync_copy(b1, o1.at[pl.ds(base, 128)])
        g2.wait()
        pltpu.sync_copy(b2, o2.at[pl.ds(base, 128)])
        g3.wait()
        pltpu.sync_copy(bt, o_t.at[pl.ds(base, 128)])

    side(idxu, e0u, e1u, e2u, utw_t, ue0, ue1, ue2, utwr)
    side(idxi, e0i, e1i, e2i, itw_t, ie0, ie1, ie2, itwr)


def _make_gather():
    out_type = [
        jax.ShapeDtypeStruct((BATCH, D), _f32),
        jax.ShapeDtypeStruct((BATCH, D), _bf16),
        jax.ShapeDtypeStruct((BATCH, D), _bf16),
        jax.ShapeDtypeStruct((BATCH, D), _f32),
        jax.ShapeDtypeStruct((BATCH, D), _bf16),
        jax.ShapeDtypeStruct((BATCH, D), _bf16),
        jax.ShapeDtypeStruct((BATCH, MTD), _f32),
        jax.ShapeDtypeStruct((BATCH, MTD), _f32),
    ]
    scratch = [
        pltpu.VMEM((128,), _i32),
        pltpu.VMEM((128,), _i32),
        pltpu.VMEM((128, D), _f32),
        pltpu.VMEM((128, D), _bf16),
        pltpu.VMEM((128, D), _bf16),
        pltpu.VMEM((128, MTD), _f32),
    ] + [pltpu.SemaphoreType.DMA] * 4
    return pl.kernel(
        lambda *refs: _gather_body(refs),
        out_type=out_type, mesh=_mesh(), scratch_types=scratch,
        compiler_params=pltpu.CompilerParams(use_tc_tiling_on_sc=False),
        name="batch_gather")


def _tc_body(ue0, ue1, ue2, ie0, ie1, ie2, utw, itw, utr, itr,
             w1, b1, w2, b2, o1, o2, oreg):
    third = _f32(1.0 / 3.0)
    ufv = (ue0[...] + ue1[...].astype(_f32) + ue2[...].astype(_f32)) * third
    ifv = (ie0[...] + ie1[...].astype(_f32) + ie2[...].astype(_f32)) * third
    g = jnp.sum(ufv * ifv, axis=1)
    o1[...] = jax.nn.sigmoid(g)
    utrv = utr[...]
    itrv = itr[...]
    w1v = w1[...]
    b1v = b1[...]
    w2v = w2[...]
    b2v = b2[...]
    hu = jnp.maximum(jnp.dot(utrv, w1v, preferred_element_type=_f32) + b1v[None, :], 0.0)
    mu = jnp.dot(hu, w2v, preferred_element_type=_f32) + b2v[None, :]
    hi = jnp.maximum(jnp.dot(itrv, w1v, preferred_element_type=_f32) + b1v[None, :], 0.0)
    mi = jnp.dot(hi, w2v, preferred_element_type=_f32) + b2v[None, :]
    utwv = utw[...]
    itwv = itw[...]
    tm = (jnp.sum(utwv[:, :TD] * utrv, axis=1) + jnp.sum(utwv[:, TD:] * mu, axis=1)
          + jnp.sum(itwv[:, :TD] * itrv, axis=1) + jnp.sum(itwv[:, TD:] * mi, axis=1))
    o2[...] = jax.nn.sigmoid(tm)
    reg = 0.5 * (jnp.sum(ue0[...] ** 2) + jnp.sum(ie0[...] ** 2)
                 + jnp.sum(utwv ** 2) + jnp.sum(itwv ** 2)) / float(BATCH)
    oreg[...] = jnp.full((1, 128), reg, _f32)


def kernel(user_indices, item_indices, time_diffs, user_trends, item_trends,
           edge_index, user_emb_w, item_emb_w, user_time_w, item_time_w,
           mlp_w1, mlp_b1, mlp_w2, mlp_b2):
    ei = edge_index.astype(_i32)
    u = ei[0, :EH]
    i = ei[0, EH:] - NU

    def pad4(x, fill):
        xr = x.reshape(NS, EPS)
        xp = jnp.pad(xr, ((0, 0), (0, ESP - EPS)), constant_values=fill)
        return xp.reshape(NS, NG, GW, CW)

    u4 = pad4(u, DUMMY)
    i4 = pad4(i, DUMMY)
    dstu4, srci4 = u4, i4
    dsti4, srcu4 = i4, u4

    def pad_tab16(w):
        wp = jnp.pad(w.astype(_bf16), ((0, RT - NU), (0, 0)))
        return wp

    tabu16 = pad_tab16(user_emb_w)
    tabi16 = pad_tab16(item_emb_w)

    e1u, e1i, invu, invi = _make_layer_a()(
        dstu4, srci4, dsti4, srcu4, tabu16, tabi16)
    e2u, e2i = _make_layer_b()(
        dstu4, srci4, dsti4, srcu4, e1u, e1i, invu, invi)
    ue0, ue1, ue2, ie0, ie1, ie2, utwr, itwr = _make_gather()(
        user_indices.astype(_i32), item_indices.astype(_i32),
        user_emb_w, e1u, e2u, item_emb_w, e1i, e2i, user_time_w, item_time_w)

    o1, o2, oreg = pl.pallas_call(
        _tc_body,
        out_shape=[
            jax.ShapeDtypeStruct((BATCH,), _f32),
            jax.ShapeDtypeStruct((BATCH,), _f32),
            jax.ShapeDtypeStruct((1, 128), _f32),
        ],
    )(ue0, ue1, ue2, ie0, ie1, ie2, utwr, itwr, user_trends, item_trends,
      mlp_w1, mlp_b1, mlp_w2, mlp_b2)
    return (o1, o2, oreg[0, 0])


# speedup vs baseline: 1.0035x; 1.0035x over previous
import jax
import jax.numpy as jnp
from jax import lax
from jax.experimental import pallas as pl
from jax.experimental.pallas import tpu as pltpu
from jax.experimental.pallas import tpu_sc as plsc

NU = 25000
NI = 25000
D = 64
EH = 400000
NS = 16
NC = 2
CW = 112
EPS = EH // NS
GW = 8
NG = 28
NJ = NG * GW
ESP = NJ * CW
RS = 1568
RT = NS * RS
EB = 112
DUMMY = NU
NRB = 4
BATCH = 4096
TD = 64
MTD = 128

_f32 = jnp.float32
_bf16 = jnp.bfloat16
_i32 = jnp.int32


def _mesh():
    return plsc.VectorSubcoreMesh(core_axis_name="c", subcore_axis_name="s")


def _layer_body(with_deg, refs):
    if with_deg:
        (dstu4, srci4, dsti4, srcu4, tab_u, tab_i,
         out_u, out_i, inv_u, inv_i,
         dw0, dw1, sw0, sw1, r0, r1, r2, r3, invblk, tabsp, acc, degacc,
         sid0, sid1, sis0, sis1, sg0, sg1, sg2, sg3, ss0, ss1, ss2, ss3,
         sd0, sd1, sd2, sd3) = refs
        invin_u = invin_i = None
    else:
        (dstu4, srci4, dsti4, srcu4, tab_u, tab_i, invin_u, invin_i,
         out_u, out_i,
         dw0, dw1, sw0, sw1, r0, r1, r2, r3, invblk, tabsp, acc, degacc,
         sid0, sid1, sis0, sis1, sg0, sg1, sg2, sg3, ss0, ss1, ss2, ss3,
         sd0, sd1, sd2, sd3) = refs
        inv_u = inv_i = None
    dw = [dw0, dw1]
    sw = [sw0, sw1]
    rows = [r0, r1, r2, r3]
    sidx = [sid0, sid1]
    sis = [sis0, sis1]
    sg = [sg0, sg1, sg2, sg3]
    ss = [ss0, ss1, ss2, ss3]
    sd = [sd0, sd1, sd2, sd3]

    cid = lax.axis_index("c")
    sid = lax.axis_index("s")

    def half(dst4, src4, table, out_e, inv_out, inv_in):
        base = sid * RS
        z32 = jnp.zeros((32,), _bf16)
        zf = jnp.zeros((16,), _f32)

        pltpu.sync_copy(table.at[pl.ds(base, RS)], tabsp.at[pl.ds(base, RS)])

        @pl.loop(0, EB)
        def _(r):
            for c in range(D // 32):
                rows[0][r, pl.ds(c * 32, 32)] = z32

        for c in range(128 // 16):
            invblk[pl.ds(c * 16, 16)] = zf

        for k in range(RS // EB):
            pltpu.sync_copy(rows[0], acc.at[pl.ds(base + k * EB, EB)])
        if with_deg:
            for k in range(RS // EB):
                pltpu.sync_copy(invblk.at[pl.ds(0, EB)],
                                degacc.at[pl.ds(base + k * EB, EB)])
            for c in range(CW // 16):
                invblk[pl.ds(c * 16, 16)] = jnp.full((16,), 1.0, _f32)

        plsc.subcore_barrier()

        def idx_start(g, b):
            pltpu.async_copy(dst4.at[sid, g], dw[b], sidx[b])
            pltpu.async_copy(src4.at[sid, g], sw[b], sis[b])

        def idx_wait(b):
            pltpu.make_async_copy(dst4.at[sid, 0], dw[b], sidx[b]).wait()
            pltpu.make_async_copy(src4.at[sid, 0], sw[b], sis[b]).wait()

        def gather_start(wb, jj, b):
            pltpu.async_copy(tabsp.at[sw[wb].at[jj]], rows[b], sg[b])

        def gather_wait(b):
            pltpu.make_async_copy(tabsp.at[pl.ds(0, CW)], rows[b],
                                  sg[b]).wait()

        def scat_start(wb, jj, b):
            pltpu.async_copy(rows[b], acc.at[dw[wb].at[jj]], ss[b], add=True)
            if with_deg:
                pltpu.async_copy(invblk.at[pl.ds(0, CW)],
                                 degacc.at[dw[wb].at[jj]], sd[b], add=True)

        def scat_wait(b):
            pltpu.make_async_copy(rows[b], acc.at[pl.ds(0, CW)],
                                  ss[b]).wait()
            if with_deg:
                pltpu.make_async_copy(invblk.at[pl.ds(0, CW)],
                                      degacc.at[pl.ds(0, CW)], sd[b]).wait()

        def do_group(wb, first, prefetch):
            idx_wait(wb)
            for p in range(3):
                if not first:
                    scat_wait(p)
                gather_start(wb, p, p)
            for jj in range(GW):
                b = jj % NRB
                gather_wait(b)
                scat_start(wb, jj, b)
                if jj + 3 < GW:
                    bn = (jj + 3) % NRB
                    if not (first and jj < 1):
                        scat_wait(bn)
                    if jj == 0:
                        prefetch()
                    gather_start(wb, jj + 3, bn)

        idx_start(0, 0)
        do_group(0, True, lambda: idx_start(1, 1))
        do_group(1, False, lambda: idx_start(2, 0))

        @pl.loop(1, NG // 2)
        def _(t):
            def pf_a():
                idx_start(2 * t + 1, 1)

            def pf_b():
                @pl.when(2 * t + 2 < NG)
                def _():
                    idx_start(2 * t + 2, 0)

            do_group(0, False, pf_a)
            do_group(1, False, pf_b)

        for b in range(NRB):
            scat_wait(b)

        plsc.subcore_barrier()

        for blk in range(RS // EB):
            off = base + blk * EB
            if with_deg:
                pltpu.sync_copy(degacc.at[pl.ds(off, EB)],
                                invblk.at[pl.ds(0, EB)])

                @pl.loop(0, EB // 16)
                def _(k):
                    v = invblk[pl.ds(k * 16, 16)]
                    invblk[pl.ds(k * 16, 16)] = 1.0 / jnp.maximum(v, 1.0)

                pltpu.sync_copy(invblk.at[pl.ds(0, EB)],
                                inv_out.at[pl.ds(off, EB)])
            else:
                pltpu.sync_copy(inv_in.at[pl.ds(off, EB)],
                                invblk.at[pl.ds(0, EB)])

            pltpu.sync_copy(acc.at[pl.ds(off, EB)], rows[0])

            @pl.loop(0, EB // 16)
            def _(k):
                dv = invblk[pl.ds(k * 16, 16)]
                for q in range(16):
                    r = k * 16 + q
                    vs = jnp.full((16,), dv[q], _f32)
                    sb = plsc.pack(vs, vs, format=plsc.PackFormat.INTERLEAVED)
                    for c in range(D // 32):
                        sl = pl.ds(c * 32, 32)
                        rows[0][r, sl] = rows[0][r, sl] * sb

            pltpu.sync_copy(rows[0], out_e.at[pl.ds(off, EB)])

    @pl.when(cid == 0)
    def _():
        half(dstu4, srci4, tab_i, out_u, inv_u, invin_u)

    @pl.when(cid == 1)
    def _():
        half(dsti4, srcu4, tab_u, out_i, inv_i, invin_i)


def _layer_scratch():
    return [
        pltpu.VMEM((GW, CW), _i32),
        pltpu.VMEM((GW, CW), _i32),
        pltpu.VMEM((GW, CW), _i32),
        pltpu.VMEM((GW, CW), _i32),
        pltpu.VMEM((EB, D), _bf16),
        pltpu.VMEM((EB, D), _bf16),
        pltpu.VMEM((EB, D), _bf16),
        pltpu.VMEM((EB, D), _bf16),
        pltpu.VMEM((128,), _f32),
        pltpu.VMEM_SHARED((RT, D), _bf16),
        pltpu.VMEM_SHARED((RT, D), _bf16),
        pltpu.VMEM_SHARED((RT,), _f32),
    ] + [pltpu.SemaphoreType.DMA] * 16


def _make_layer_a():
    out_type = [
        jax.ShapeDtypeStruct((RT, D), _bf16),
        jax.ShapeDtypeStruct((RT, D), _bf16),
        jax.ShapeDtypeStruct((RT,), _f32),
        jax.ShapeDtypeStruct((RT,), _f32),
    ]
    return pl.kernel(
        lambda *refs: _layer_body(True, refs),
        out_type=out_type, mesh=_mesh(), scratch_types=_layer_scratch(),
        compiler_params=pltpu.CompilerParams(use_tc_tiling_on_sc=False,
                                             needs_layout_passes=False),
        name="gnn_layer1")


def _make_layer_b():
    out_type = [
        jax.ShapeDtypeStruct((RT, D), _bf16),
        jax.ShapeDtypeStruct((RT, D), _bf16),
    ]
    return pl.kernel(
        lambda *refs: _layer_body(False, refs),
        out_type=out_type, mesh=_mesh(), scratch_types=_layer_scratch(),
        compiler_params=pltpu.CompilerParams(use_tc_tiling_on_sc=False,
                                             needs_layout_passes=False),
        name="gnn_layer2")


def _gather_body(refs):
    (u_idx, i_idx, e0u, e1u, e2u, e0i, e1i, e2i, utw_t, itw_t,
     ue0, ue1, ue2, ie0, ie1, ie2, utwr, itwr,
     idxu, idxi, b0, b1, b2, bt, s0, s1, s2, s3) = refs
    cid = lax.axis_index("c")
    sid = lax.axis_index("s")
    wid = cid * NS + sid
    base = wid * 128
    pltpu.sync_copy(u_idx.at[pl.ds(base, 128)], idxu)
    pltpu.sync_copy(i_idx.at[pl.ds(base, 128)], idxi)

    def side(idxv, t0, t1, t2, tt, o0, o1, o2, o_t):
        g0 = pltpu.async_copy(t0.at[idxv], b0, s0)
        g1 = pltpu.async_copy(t1.at[idxv], b1, s1)
        g2 = pltpu.async_copy(t2.at[idxv], b2, s2)
        g3 = pltpu.async_copy(tt.at[idxv], bt, s3)
        g0.wait()
        pltpu.sync_copy(b0, o0.at[pl.ds(base, 128)])
        g1.wait()
        pltpu.sync_copy(b1, o1.at[pl.ds(base, 128)])
        g2.wait()
        pltpu.sync_copy(b2, o2.at[pl.ds(base, 128)])
        g3.wait()
        pltpu.sync_copy(bt, o_t.at[pl.ds(base, 128)])

    side(idxu, e0u, e1u, e2u, utw_t, ue0, ue1, ue2, utwr)
    side(idxi, e0i, e1i, e2i, itw_t, ie0, ie1, ie2, itwr)


def _make_gather():
    out_type = [
        jax.ShapeDtypeStruct((BATCH, D), _f32),
        jax.ShapeDtypeStruct((BATCH, D), _bf16),
        jax.ShapeDtypeStruct((BATCH, D), _bf16),
        jax.ShapeDtypeStruct((BATCH, D), _f32),
        jax.ShapeDtypeStruct((BATCH, D), _bf16),
        jax.ShapeDtypeStruct((BATCH, D), _bf16),
        jax.ShapeDtypeStruct((BATCH, MTD), _f32),
        jax.ShapeDtypeStruct((BATCH, MTD), _f32),
    ]
    scratch = [
        pltpu.VMEM((128,), _i32),
        pltpu.VMEM((128,), _i32),
        pltpu.VMEM((128, D), _f32),
        pltpu.VMEM((128, D), _bf16),
        pltpu.VMEM((128, D), _bf16),
        pltpu.VMEM((128, MTD), _f32),
    ] + [pltpu.SemaphoreType.DMA] * 4
    return pl.kernel(
        lambda *refs: _gather_body(refs),
        out_type=out_type, mesh=_mesh(), scratch_types=scratch,
        compiler_params=pltpu.CompilerParams(use_tc_tiling_on_sc=False),
        name="batch_gather")


def _tc_body(ue0, ue1, ue2, ie0, ie1, ie2, utw, itw, utr, itr,
             w1, b1, w2, b2, o1, o2, oreg):
    third = _f32(1.0 / 3.0)
    ufv = (ue0[...] + ue1[...].astype(_f32) + ue2[...].astype(_f32)) * third
    ifv = (ie0[...] + ie1[...].astype(_f32) + ie2[...].astype(_f32)) * third
    g = jnp.sum(ufv * ifv, axis=1)
    o1[...] = jax.nn.sigmoid(g)
    utrv = utr[...]
    itrv = itr[...]
    w1v = w1[...]
    b1v = b1[...]
    w2v = w2[...]
    b2v = b2[...]
    hu = jnp.maximum(jnp.dot(utrv, w1v, preferred_element_type=_f32) + b1v[None, :], 0.0)
    mu = jnp.dot(hu, w2v, preferred_element_type=_f32) + b2v[None, :]
    hi = jnp.maximum(jnp.dot(itrv, w1v, preferred_element_type=_f32) + b1v[None, :], 0.0)
    mi = jnp.dot(hi, w2v, preferred_element_type=_f32) + b2v[None, :]
    utwv = utw[...]
    itwv = itw[...]
    tm = (jnp.sum(utwv[:, :TD] * utrv, axis=1) + jnp.sum(utwv[:, TD:] * mu, axis=1)
          + jnp.sum(itwv[:, :TD] * itrv, axis=1) + jnp.sum(itwv[:, TD:] * mi, axis=1))
    o2[...] = jax.nn.sigmoid(tm)
    reg = 0.5 * (jnp.sum(ue0[...] ** 2) + jnp.sum(ie0[...] ** 2)
                 + jnp.sum(utwv ** 2) + jnp.sum(itwv ** 2)) / float(BATCH)
    oreg[...] = jnp.full((1, 128), reg, _f32)


def kernel(user_indices, item_indices, time_diffs, user_trends, item_trends,
           edge_index, user_emb_w, item_emb_w, user_time_w, item_time_w,
           mlp_w1, mlp_b1, mlp_w2, mlp_b2):
    ei = edge_index.astype(_i32)
    u = ei[0, :EH]
    i = ei[0, EH:] - NU

    def pad4(x, fill):
        xr = x.reshape(NS, EPS)
        xp = jnp.pad(xr, ((0, 0), (0, ESP - EPS)), constant_values=fill)
        return xp.reshape(NS, NG, GW, CW)

    u4 = pad4(u, DUMMY)
    i4 = pad4(i, DUMMY)
    dstu4, srci4 = u4, i4
    dsti4, srcu4 = i4, u4

    def pad_tab16(w):
        wp = jnp.pad(w.astype(_bf16), ((0, RT - NU), (0, 0)))
        return wp

    tabu16 = pad_tab16(user_emb_w)
    tabi16 = pad_tab16(item_emb_w)

    e1u, e1i, invu, invi = _make_layer_a()(
        dstu4, srci4, dsti4, srcu4, tabu16, tabi16)
    e2u, e2i = _make_layer_b()(
        dstu4, srci4, dsti4, srcu4, e1u, e1i, invu, invi)
    ue0, ue1, ue2, ie0, ie1, ie2, utwr, itwr = _make_gather()(
        user_indices.astype(_i32), item_indices.astype(_i32),
        user_emb_w, e1u, e2u, item_emb_w, e1i, e2i, user_time_w, item_time_w)

    o1, o2, oreg = pl.pallas_call(
        _tc_body,
        out_shape=[
            jax.ShapeDtypeStruct((BATCH,), _f32),
            jax.ShapeDtypeStruct((BATCH,), _f32),
            jax.ShapeDtypeStruct((1, 128), _f32),
        ],
    )(ue0, ue1, ue2, ie0, ie1, ie2, utwr, itwr, user_trends, item_trends,
      mlp_w1, mlp_b1, mlp_w2, mlp_b2)
    return (o1, o2, oreg[0, 0])
